# baseline (speedup 1.0000x reference)
import functools

import jax
import jax.numpy as jnp
from jax import lax
from jax.experimental import pallas as pl
from jax.experimental.pallas import tpu as pltpu
from jax.experimental.pallas import tpu_sc as plsc


def _sc_gather(user_id, time, day_of_week, user_table, time_table, dow_table):
    B = user_id.shape[0]
    EU = user_table.shape[1]
    ET = time_table.shape[1]
    info = plsc.get_sparse_core_info()
    NW = info.num_cores * info.num_subcores
    b_per_w = B // NW
    mesh = plsc.VectorSubcoreMesh(core_axis_name="c", subcore_axis_name="s")

    @functools.partial(
        pl.kernel,
        mesh=mesh,
        compiler_params=pltpu.CompilerParams(use_tc_tiling_on_sc=False),
        out_type=(
            jax.ShapeDtypeStruct((B, EU), jnp.float32),
            jax.ShapeDtypeStruct((B, ET), jnp.float32),
            jax.ShapeDtypeStruct((B, ET), jnp.float32),
        ),
        scratch_types=[
            pltpu.VMEM((b_per_w,), jnp.int32),
            pltpu.VMEM((b_per_w,), jnp.int32),
            pltpu.VMEM((b_per_w,), jnp.int32),
            pltpu.VMEM((b_per_w, EU), jnp.float32),
            pltpu.VMEM((b_per_w, ET), jnp.float32),
            pltpu.VMEM((b_per_w, ET), jnp.float32),
            pltpu.SemaphoreType.DMA,
            pltpu.SemaphoreType.DMA,
            pltpu.SemaphoreType.DMA,
        ],
    )
    def gather_kernel(uid_hbm, t_hbm, d_hbm, ut_hbm, tt_hbm, dt_hbm,
                      out_u, out_t, out_d,
                      idx_u, idx_t, idx_d, rows_u, rows_t, rows_d,
                      sem_u, sem_t, sem_d):
        wid = lax.axis_index("s") * info.num_cores + lax.axis_index("c")
        base = wid * b_per_w
        pltpu.sync_copy(uid_hbm.at[pl.ds(base, b_per_w)], idx_u)
        pltpu.sync_copy(t_hbm.at[pl.ds(base, b_per_w)], idx_t)
        pltpu.sync_copy(d_hbm.at[pl.ds(base, b_per_w)], idx_d)
        cu = pltpu.async_copy(ut_hbm.at[idx_u], rows_u, sem_u)
        ct = pltpu.async_copy(tt_hbm.at[idx_t], rows_t, sem_t)
        cd = pltpu.async_copy(dt_hbm.at[idx_d], rows_d, sem_d)
        cu.wait()
        ct.wait()
        cd.wait()
        pltpu.sync_copy(rows_u, out_u.at[pl.ds(base, b_per_w)])
        pltpu.sync_copy(rows_t, out_t.at[pl.ds(base, b_per_w)])
        pltpu.sync_copy(rows_d, out_d.at[pl.ds(base, b_per_w)])

    return gather_kernel(user_id, time, day_of_week,
                         user_table, time_table, dow_table)


def _tc_project(u, t, d, W, b2):
    B, EU = u.shape
    ET = t.shape[1]
    N = W.shape[1]
    BM = 512

    def body(u_ref, t_ref, d_ref, w_ref, b_ref, o_ref):
        acc = jnp.dot(u_ref[...], w_ref[0:EU, :],
                      preferred_element_type=jnp.float32)
        acc += jnp.dot(t_ref[...], w_ref[EU:EU + ET, :],
                       preferred_element_type=jnp.float32)
        acc += jnp.dot(d_ref[...], w_ref[EU + ET:EU + 2 * ET, :],
                       preferred_element_type=jnp.float32)
        o_ref[...] = acc + b_ref[...]

    return pl.pallas_call(
        body,
        grid=(B // BM,),
        in_specs=[
            pl.BlockSpec((BM, EU), lambda i: (i, 0)),
            pl.BlockSpec((BM, ET), lambda i: (i, 0)),
            pl.BlockSpec((BM, ET), lambda i: (i, 0)),
            pl.BlockSpec(W.shape, lambda i: (0, 0)),
            pl.BlockSpec((1, N), lambda i: (0, 0)),
        ],
        out_specs=pl.BlockSpec((BM, N), lambda i: (i, 0)),
        out_shape=jax.ShapeDtypeStruct((B, N), jnp.float32),
    )(u, t, d, W, b2)


def kernel(user_id, time, day_of_week, user_table, time_table, dow_table, W, b):
    u, t, d = _sc_gather(user_id, time, day_of_week,
                         user_table, time_table, dow_table)
    return _tc_project(u, t, d, W, b.reshape(1, -1))


# speedup vs baseline: 1.7701x; 1.7701x over previous
import functools

import jax
import jax.numpy as jnp
from jax import lax
from jax.experimental import pallas as pl
from jax.experimental.pallas import tpu as pltpu
from jax.experimental.pallas import tpu_sc as plsc


def _sc_gather_users(user_id, user_table):
    B = user_id.shape[0]
    E = user_table.shape[1]
    info = plsc.get_sparse_core_info()
    NW = info.num_cores * info.num_subcores
    bpw = B // NW
    mesh = plsc.VectorSubcoreMesh(core_axis_name="c", subcore_axis_name="s")

    @functools.partial(
        pl.kernel,
        mesh=mesh,
        compiler_params=pltpu.CompilerParams(use_tc_tiling_on_sc=True),
        out_type=jax.ShapeDtypeStruct((B, E), jnp.float32),
        scratch_types=[
            pltpu.VMEM((bpw,), jnp.int32),
            pltpu.VMEM((bpw, E), jnp.float32),
            pltpu.SemaphoreType.DMA,
            pltpu.SemaphoreType.DMA,
        ],
    )
    def gather_kernel(uid_hbm, table_hbm, out_hbm, idx_v, rows_v,
                      sem_i, sem_g):
        wid = lax.axis_index("s") * info.num_cores + lax.axis_index("c")
        base = wid * bpw
        pltpu.async_copy(uid_hbm.at[pl.ds(base, bpw)], idx_v, sem_i).wait()
        copies = []
        for c in range(bpw // 16):
            vec = idx_v[pl.ds(c * 16, 16)]
            for j in range(16):
                i = c * 16 + j
                copies.append(pltpu.async_copy(
                    table_hbm.at[pl.ds(vec[j], 1)],
                    rows_v.at[pl.ds(i, 1)], sem_g))
        for c in copies:
            c.wait()
        pltpu.sync_copy(rows_v, out_hbm.at[pl.ds(base, bpw)])

    return gather_kernel(user_id, user_table)


def _tc_combine(u, time2, dow2, time_table, dow_table, W, b2):
    B, EU = u.shape
    TV, ET = time_table.shape
    DV = dow_table.shape[0]
    N = W.shape[1]
    BM = 512

    def body(u_ref, t_ref, d_ref, tt_ref, dt_ref, w_ref, b_ref, o_ref):
        t_oh = (lax.broadcasted_iota(jnp.int32, (BM, TV), 1)
                == t_ref[...]).astype(jnp.float32)
        d_oh = (lax.broadcasted_iota(jnp.int32, (BM, DV), 1)
                == d_ref[...]).astype(jnp.float32)
        t_emb = jnp.dot(t_oh, tt_ref[...], preferred_element_type=jnp.float32)
        d_emb = jnp.dot(d_oh, dt_ref[...], preferred_element_type=jnp.float32)
        acc = jnp.dot(u_ref[...], w_ref[0:EU, :],
                      preferred_element_type=jnp.float32)
        acc += jnp.dot(t_emb, w_ref[EU:EU + ET, :],
                       preferred_element_type=jnp.float32)
        acc += jnp.dot(d_emb, w_ref[EU + ET:EU + 2 * ET, :],
                       preferred_element_type=jnp.float32)
        o_ref[...] = acc + b_ref[...]

    return pl.pallas_call(
        body,
        grid=(B // BM,),
        in_specs=[
            pl.BlockSpec((BM, EU), lambda i: (i, 0)),
            pl.BlockSpec((BM, 1), lambda i: (i, 0)),
            pl.BlockSpec((BM, 1), lambda i: (i, 0)),
            pl.BlockSpec((TV, ET), lambda i: (0, 0)),
            pl.BlockSpec((DV, ET), lambda i: (0, 0)),
            pl.BlockSpec(W.shape, lambda i: (0, 0)),
            pl.BlockSpec((1, N), lambda i: (0, 0)),
        ],
        out_specs=pl.BlockSpec((BM, N), lambda i: (i, 0)),
        out_shape=jax.ShapeDtypeStruct((B, N), jnp.float32),
    )(u, time2, dow2, time_table, dow_table, W, b2)


def kernel(user_id, time, day_of_week, user_table, time_table, dow_table, W, b):
    u = _sc_gather_users(user_id, user_table)
    return _tc_combine(u, time.reshape(-1, 1), day_of_week.reshape(-1, 1),
                       time_table, dow_table, W, b.reshape(1, -1))
